# baseline (speedup 1.0000x reference)
import jax
import jax.numpy as jnp
from jax import lax
from jax.experimental import pallas as pl
from jax.experimental.pallas import tpu as pltpu
from jax.experimental.pallas import tpu_sc as plsc

_NC = 2
_NS = 16
_NW = _NC * _NS

_B = 4096
_L = 20
_D = 300
_BPW = _B // _NW
_EG = 8
_NG = _BPW // _EG
_RG = _EG * _L
_DH = _D // 2
_CHUNK_OFFS = tuple(j * 16 for j in range(_DH // 16)) + (_DH - 16,)
_NCH = len(_CHUNK_OFFS)
_INV_L = 1.0 / _L


def _pool_body(x_hbm, tab_hbm, out_hbm, xv, rows0, rows1, outv, sem0, sem1):
    wid = lax.axis_index("s") * _NC + lax.axis_index("c")
    base = wid * _BPW
    pltpu.sync_copy(x_hbm.at[pl.ds(base * _L, _BPW * _L)], xv)

    def issue_group(g, rows, sem):
        for v in range(_RG // 16):
            off = pl.multiple_of(g * _RG + v * 16, 8)
            vec = xv[pl.ds(off, 16)]
            for k in range(16):
                pltpu.async_copy(tab_hbm.at[vec[k]], rows.at[v * 16 + k], sem)

    def drain_group(rows, sem):
        pltpu.make_async_copy(tab_hbm.at[pl.ds(0, _RG)], rows, sem).wait()

    def accumulate(rows, g):
        for e in range(_EG):
            def rbody(l, accs, e=e):
                r = e * _L + l
                out = []
                for j in range(_NCH):
                    w = rows[r, pl.ds(_CHUNK_OFFS[j], 16)]
                    wb = plsc.bitcast(w, jnp.bfloat16)
                    a, b = plsc.unpack(wb, format=plsc.PackFormat.INTERLEAVED,
                                       preferred_element_type=jnp.float32)
                    out.append((accs[j][0] + a, accs[j][1] + b))
                return tuple(out)
            z = jnp.zeros((16,), jnp.float32)
            accs = lax.fori_loop(0, _L, rbody,
                                 tuple((z, z) for _ in range(_NCH)))
            row_out = g * _EG + e
            for j in range(_NCH):
                packed = plsc.pack(accs[j][0] * _INV_L, accs[j][1] * _INV_L,
                                   format=plsc.PackFormat.INTERLEAVED)
                outv[row_out, pl.ds(_CHUNK_OFFS[j], 16)] = plsc.bitcast(
                    packed, jnp.float32)

    issue_group(0, rows0, sem0)
    issue_group(1, rows1, sem1)

    def gbody(g2, carry):
        g = g2 * 2
        drain_group(rows0, sem0)
        accumulate(rows0, g)
        issue_group(g + 2, rows0, sem0)
        drain_group(rows1, sem1)
        accumulate(rows1, g + 1)
        issue_group(g + 3, rows1, sem1)
        return carry

    lax.fori_loop(0, _NG // 2 - 1, gbody, 0)

    g_last = _NG - 2
    drain_group(rows0, sem0)
    accumulate(rows0, g_last)
    drain_group(rows1, sem1)
    accumulate(rows1, g_last + 1)

    pltpu.sync_copy(outv, out_hbm.at[pl.ds(base, _BPW)])


_pool = pl.kernel(
    _pool_body,
    out_type=jax.ShapeDtypeStruct((_B, _DH), jnp.float32),
    mesh=plsc.VectorSubcoreMesh(
        core_axis_name="c", subcore_axis_name="s",
        num_cores=_NC, num_subcores=_NS),
    scratch_types=[
        pltpu.VMEM((_BPW * _L,), jnp.int32),
        pltpu.VMEM((_RG, _DH), jnp.float32),
        pltpu.VMEM((_RG, _DH), jnp.float32),
        pltpu.VMEM((_BPW, _DH), jnp.float32),
        pltpu.SemaphoreType.DMA,
        pltpu.SemaphoreType.DMA,
    ],
    compiler_params=pltpu.CompilerParams(use_tc_tiling_on_sc=True,
                                         needs_layout_passes=False),
)


_TBK = 12800


def _tr_body(src_ref, dst_ref):
    packed = pltpu.bitcast(src_ref[...].astype(jnp.bfloat16), jnp.float32)
    dst_ref[...] = packed.T


def _transpose_table(table):
    tab_t = table.T
    return pl.pallas_call(
        _tr_body,
        out_shape=jax.ShapeDtypeStruct((table.shape[0], _DH), jnp.float32),
        grid=((table.shape[0] + _TBK - 1) // _TBK,),
        in_specs=[pl.BlockSpec((_D, _TBK), lambda i: (0, i))],
        out_specs=pl.BlockSpec((_TBK, _DH), lambda i: (i, 0)),
    )(tab_t)


def _mlp_body(h_ref, w1_ref, b1_ref, w2_ref, b2_ref, w3_ref, b3_ref, o_ref):
    h = pltpu.bitcast(h_ref[...].T, jnp.bfloat16).T
    h1 = jnp.maximum(
        jnp.dot(h, w1_ref[...], preferred_element_type=jnp.float32)
        + b1_ref[...], 0.0)
    h2 = jnp.maximum(
        jnp.dot(h1, w2_ref[...], preferred_element_type=jnp.float32)
        + b2_ref[...], 0.0)
    o_ref[...] = (
        jnp.dot(h2, w3_ref[...], preferred_element_type=jnp.float32)
        + b3_ref[...])


_MLP_BLK = 512


def _mlp(pooled, W1, b1, W2, b2, W3, b3):
    grid = (_B // _MLP_BLK,)
    return pl.pallas_call(
        _mlp_body,
        out_shape=jax.ShapeDtypeStruct((_B, 1), jnp.float32),
        grid=grid,
        in_specs=[
            pl.BlockSpec((_MLP_BLK, _DH), lambda i: (i, 0)),
            pl.BlockSpec(W1.shape, lambda i: (0, 0)),
            pl.BlockSpec(b1.shape, lambda i: (0, 0)),
            pl.BlockSpec(W2.shape, lambda i: (0, 0)),
            pl.BlockSpec(b2.shape, lambda i: (0, 0)),
            pl.BlockSpec(W3.shape, lambda i: (0, 0)),
            pl.BlockSpec(b3.shape, lambda i: (0, 0)),
        ],
        out_specs=pl.BlockSpec((_MLP_BLK, 1), lambda i: (i, 0)),
    )(pooled, W1, b1, W2, b2, W3, b3)


def kernel(x, table, W1, b1, W2, b2, W3, b3):
    x_flat = x.reshape(-1).astype(jnp.int32)
    table_vm = _transpose_table(table)
    pooled = _pool(x_flat, table_vm)
    return _mlp(pooled, W1, b1.reshape(1, -1), W2, b2.reshape(1, -1),
                W3, b3.reshape(1, -1))


# speedup vs baseline: 1.1115x; 1.1115x over previous
import jax
import jax.numpy as jnp
from jax import lax
from jax.experimental import pallas as pl
from jax.experimental.pallas import tpu as pltpu
from jax.experimental.pallas import tpu_sc as plsc

_NC = 2
_NS = 16
_NW = _NC * _NS

_B = 4096
_L = 20
_D = 300
_BPW = _B // _NW
_EG = 4
_NG = _BPW // _EG
_RG = _EG * _L
_DH = _D // 2
_DHA = 76
_DHB = _DH - _DHA
_INV_L = 1.0 / _L


def _chunk_offs(dh):
    offs = tuple(j * 16 for j in range(dh // 16))
    if dh % 16:
        offs = offs + (dh - 16,)
    return offs


def _pool_body(x_hbm, tab_hbm, out_hbm, xv, rows0, rows1, outv, sem0, sem1,
               dh):
    offs = _chunk_offs(dh)
    nch = len(offs)
    wid = lax.axis_index("s") * _NC + lax.axis_index("c")
    base = wid * _BPW
    pltpu.sync_copy(x_hbm.at[pl.ds(base * _L, _BPW * _L)], xv)

    def issue_group(g, rows, sem):
        for v in range(_RG // 16):
            off = pl.multiple_of(g * _RG + v * 16, 8)
            vec = xv[pl.ds(off, 16)]
            for k in range(16):
                pltpu.async_copy(tab_hbm.at[vec[k]], rows.at[v * 16 + k], sem)

    def drain_group(rows, sem):
        pltpu.make_async_copy(tab_hbm.at[pl.ds(0, _RG)], rows, sem).wait()

    def accumulate(rows, g):
        for e in range(_EG):
            def rbody(l, accs, e=e):
                r = e * _L + l
                out = []
                for j in range(nch):
                    w = rows[r, pl.ds(offs[j], 16)]
                    wb = plsc.bitcast(w, jnp.bfloat16)
                    a, b = plsc.unpack(wb, format=plsc.PackFormat.INTERLEAVED,
                                       preferred_element_type=jnp.float32)
                    out.append((accs[j][0] + a, accs[j][1] + b))
                return tuple(out)
            z = jnp.zeros((16,), jnp.float32)
            accs = lax.fori_loop(0, _L, rbody,
                                 tuple((z, z) for _ in range(nch)))
            row_out = g * _EG + e
            for j in range(nch):
                packed = plsc.pack(accs[j][0] * _INV_L, accs[j][1] * _INV_L,
                                   format=plsc.PackFormat.INTERLEAVED)
                outv[row_out, pl.ds(offs[j], 16)] = plsc.bitcast(
                    packed, jnp.float32)

    issue_group(0, rows0, sem0)
    issue_group(1, rows1, sem1)

    def gbody(g2, carry):
        g = g2 * 2
        drain_group(rows0, sem0)
        accumulate(rows0, g)
        issue_group(g + 2, rows0, sem0)
        drain_group(rows1, sem1)
        accumulate(rows1, g + 1)
        issue_group(g + 3, rows1, sem1)
        return carry

    lax.fori_loop(0, _NG // 2 - 1, gbody, 0)

    g_last = _NG - 2
    drain_group(rows0, sem0)
    accumulate(rows0, g_last)
    drain_group(rows1, sem1)
    accumulate(rows1, g_last + 1)

    pltpu.sync_copy(outv, out_hbm.at[pl.ds(base, _BPW)])


def _make_pool(dh):
    import functools
    return pl.kernel(
        functools.partial(_pool_body, dh=dh),
        out_type=jax.ShapeDtypeStruct((_B, dh), jnp.float32),
        mesh=plsc.VectorSubcoreMesh(
            core_axis_name="c", subcore_axis_name="s",
            num_cores=_NC, num_subcores=_NS),
        scratch_types=[
            pltpu.VMEM((_BPW * _L,), jnp.int32),
            pltpu.VMEM((_RG, dh), jnp.float32),
            pltpu.VMEM((_RG, dh), jnp.float32),
            pltpu.VMEM((_BPW, dh), jnp.float32),
            pltpu.SemaphoreType.DMA,
            pltpu.SemaphoreType.DMA,
        ],
        compiler_params=pltpu.CompilerParams(use_tc_tiling_on_sc=True,
                                             needs_layout_passes=False),
    )


_pool_a = _make_pool(_DHA)
_pool_b = _make_pool(_DHB)


_TBK = 12800


def _make_tr_body(dh):
    def _tr_body(src_ref, dst_ref):
        packed = pltpu.bitcast(src_ref[...].astype(jnp.bfloat16), jnp.float32)
        dst_ref[...] = packed[:dh].T
    return _tr_body


def _transpose_half(table, half, dh):
    tab_t = table.T
    return pl.pallas_call(
        _make_tr_body(dh),
        out_shape=jax.ShapeDtypeStruct((table.shape[0], dh), jnp.float32),
        grid=((table.shape[0] + _TBK - 1) // _TBK,),
        in_specs=[pl.BlockSpec((2 * _DHA, _TBK), lambda i, h=half: (h, i))],
        out_specs=pl.BlockSpec((_TBK, dh), lambda i: (i, 0)),
    )(tab_t)


def _mlp_body(ha_ref, hb_ref, w1_ref, b1_ref, w2_ref, b2_ref, w3_ref, b3_ref,
              o_ref):
    ha = pltpu.bitcast(ha_ref[...].T, jnp.bfloat16).T
    hb = pltpu.bitcast(hb_ref[...].T, jnp.bfloat16).T
    h = jnp.concatenate([ha, hb], axis=1)
    h1 = jnp.maximum(
        jnp.dot(h, w1_ref[...], preferred_element_type=jnp.float32)
        + b1_ref[...], 0.0)
    h2 = jnp.maximum(
        jnp.dot(h1, w2_ref[...], preferred_element_type=jnp.float32)
        + b2_ref[...], 0.0)
    o_ref[...] = (
        jnp.dot(h2, w3_ref[...], preferred_element_type=jnp.float32)
        + b3_ref[...])


_MLP_BLK = 512


def _mlp(pooled_a, pooled_b, W1, b1, W2, b2, W3, b3):
    grid = (_B // _MLP_BLK,)
    return pl.pallas_call(
        _mlp_body,
        out_shape=jax.ShapeDtypeStruct((_B, 1), jnp.float32),
        grid=grid,
        in_specs=[
            pl.BlockSpec((_MLP_BLK, _DHA), lambda i: (i, 0)),
            pl.BlockSpec((_MLP_BLK, _DHB), lambda i: (i, 0)),
            pl.BlockSpec(W1.shape, lambda i: (0, 0)),
            pl.BlockSpec(b1.shape, lambda i: (0, 0)),
            pl.BlockSpec(W2.shape, lambda i: (0, 0)),
            pl.BlockSpec(b2.shape, lambda i: (0, 0)),
            pl.BlockSpec(W3.shape, lambda i: (0, 0)),
            pl.BlockSpec(b3.shape, lambda i: (0, 0)),
        ],
        out_specs=pl.BlockSpec((_MLP_BLK, 1), lambda i: (i, 0)),
    )(pooled_a, pooled_b, W1, b1, W2, b2, W3, b3)


def kernel(x, table, W1, b1, W2, b2, W3, b3):
    x_flat = x.reshape(-1).astype(jnp.int32)
    tab_a = _transpose_half(table, 0, _DHA)
    pooled_a = _pool_a(x_flat, tab_a)
    tab_b = _transpose_half(table, 1, _DHB)
    pooled_b = _pool_b(x_flat, tab_b)
    return _mlp(pooled_a, pooled_b, W1, b1.reshape(1, -1),
                W2, b2.reshape(1, -1), W3, b3.reshape(1, -1))
